# Initial kernel scaffold; baseline (speedup 1.0000x reference)
import jax
import jax.numpy as jnp
from jax.experimental import pallas as pl


def kernel(input_ids, weight):
    raise NotImplementedError("write your pallas kernel here")


# speedup vs baseline: 4.0381x; 4.0381x over previous
import functools

import jax
import jax.numpy as jnp
from jax import lax
from jax.experimental import pallas as pl
from jax.experimental.pallas import tpu as pltpu
from jax.experimental.pallas import tpu_sc as plsc

WORLD = 8
NC, NS = 2, 16
NW = NC * NS
CHUNK = 80
ZROWS = 512


def _make_sc_call(B, V, D):
    local_vocab = V // WORLD
    tpw = B // NW
    nchunk = tpw // CHUNK
    rows_per_w = tpw * WORLD
    nz = rows_per_w // ZROWS
    assert tpw % CHUNK == 0 and rows_per_w % ZROWS == 0

    mesh = plsc.VectorSubcoreMesh(core_axis_name="c", subcore_axis_name="s",
                                  num_cores=NC, num_subcores=NS)

    @functools.partial(
        pl.kernel,
        out_type=jax.ShapeDtypeStruct((B * WORLD, D), jnp.float32),
        mesh=mesh,
        scratch_types=[
            pltpu.VMEM((nchunk, CHUNK), jnp.int32),
            pltpu.VMEM((nchunk, CHUNK), jnp.int32),
            pltpu.VMEM((CHUNK, D), jnp.float32),
            pltpu.VMEM((ZROWS, D), jnp.float32),
            pltpu.SemaphoreType.DMA,
        ],
    )
    def sc_call(ids_hbm, table_hbm, zeros_hbm, out_hbm,
                ids_v, dst_v, rows_v, zbuf, sem):
        wid = lax.axis_index("s") * NC + lax.axis_index("c")

        pltpu.sync_copy(zeros_hbm, zbuf)
        row0 = wid * rows_per_w

        def zbody(z, carry):
            pltpu.sync_copy(zbuf, out_hbm.at[pl.ds(row0 + z * ZROWS, ZROWS)])
            return carry

        lax.fori_loop(0, nz, zbody, 0)

        pltpu.sync_copy(ids_hbm.at[wid], ids_v)
        tok0 = wid * tpw

        def cbody(c, carry):
            for g in range(CHUNK // 16):
                v = ids_v[c, pl.ds(g * 16, 16)]
                shard = lax.div(v, jnp.int32(local_vocab))
                t = (tok0 + c * CHUNK + g * 16
                     + lax.broadcasted_iota(jnp.int32, (16,), 0))
                dst_v[c, pl.ds(g * 16, 16)] = t * WORLD + shard
            pltpu.async_copy(table_hbm.at[ids_v.at[c]], rows_v, sem).wait()
            pltpu.async_copy(rows_v, out_hbm.at[dst_v.at[c]], sem).wait()
            return carry

        lax.fori_loop(0, nchunk, cbody, 0)

    return sc_call


def kernel(input_ids, weight):
    R, S = input_ids.shape
    V, D = weight.shape
    B = R * S
    ids3 = input_ids.reshape(-1).astype(jnp.int32).reshape(NW, B // NW // CHUNK, CHUNK)
    zeros = jnp.zeros((ZROWS, D), jnp.float32)
    out = _make_sc_call(B, V, D)(ids3, weight, zeros)
    return out.reshape(R, S, WORLD * D)
